# baseline (speedup 1.0000x reference)
import functools

import jax
import jax.numpy as jnp
from jax.experimental import pallas as pl
from jax.experimental.pallas import tpu as pltpu


def _body(P1, e_ref, qc_ref, cq_ref, out_ref, msg_ref):
    i = pl.program_id(0)

    @pl.when(i < P1)
    def _p1():
        msg_ref[pl.ds(i * qc_ref.shape[0], qc_ref.shape[0]), :] = jnp.dot(
            qc_ref[...], e_ref[...], preferred_element_type=jnp.float32
        )

    @pl.when(i >= P1)
    def _p2():
        out_ref[...] = jnp.dot(
            cq_ref[...], msg_ref[...], preferred_element_type=jnp.float32
        )


@jax.jit
def kernel(skill_embs, HG_qc, HG_cq):
    n_edges, n_skills = HG_qc.shape
    d = skill_embs.shape[1]
    BQ = 128
    BR = 128
    P1 = n_edges // BQ
    P2 = n_skills // BR

    return pl.pallas_call(
        functools.partial(_body, P1),
        grid=(P1 + P2,),
        in_specs=[
            pl.BlockSpec((n_skills, d), lambda i: (0, 0)),
            pl.BlockSpec((BQ, n_skills), lambda i: (jnp.minimum(i, P1 - 1), 0)),
            pl.BlockSpec((BR, n_edges), lambda i: (jnp.maximum(i - P1, 0), 0)),
        ],
        out_specs=pl.BlockSpec((BR, d), lambda i: (jnp.maximum(i - P1, 0), 0)),
        out_shape=jax.ShapeDtypeStruct((n_skills, d), jnp.float32),
        scratch_shapes=[pltpu.VMEM((n_edges, d), jnp.float32)],
    )(skill_embs, HG_qc, HG_cq)


# speedup vs baseline: 1.2338x; 1.2338x over previous
import functools

import jax
import jax.numpy as jnp
from jax.experimental import pallas as pl
from jax.experimental.pallas import tpu as pltpu

LANES = 4


def _body(P1, BQ, BR, *refs):
    e_ref = refs[0]
    qc_refs = refs[1:1 + LANES]
    cq_refs = refs[1 + LANES:1 + 2 * LANES]
    out_ref = refs[1 + 2 * LANES]
    msg_ref = refs[2 + 2 * LANES]
    i = pl.program_id(0)

    @pl.when(i < P1)
    def _p1():
        for k in range(LANES):
            msg_ref[pl.ds((i * LANES + k) * BQ, BQ), :] = jnp.dot(
                qc_refs[k][...], e_ref[...], preferred_element_type=jnp.float32
            )

    @pl.when(i >= P1)
    def _p2():
        for k in range(LANES):
            out_ref[k * BR:(k + 1) * BR, :] = jnp.dot(
                cq_refs[k][...], msg_ref[...], preferred_element_type=jnp.float32
            )


@jax.jit
def kernel(skill_embs, HG_qc, HG_cq):
    n_edges, n_skills = HG_qc.shape
    d = skill_embs.shape[1]
    BQ = 32
    BR = 128
    P1 = n_edges // (BQ * LANES)
    P2 = n_skills // (BR * LANES)

    qc_specs = [
        pl.BlockSpec(
            (BQ, n_skills),
            lambda i, k=k: (jnp.minimum(i, P1 - 1) * LANES + k, 0),
        )
        for k in range(LANES)
    ]
    cq_specs = [
        pl.BlockSpec(
            (BR, n_edges),
            lambda i, k=k: (jnp.maximum(i - P1, 0) * LANES + k, 0),
        )
        for k in range(LANES)
    ]

    return pl.pallas_call(
        functools.partial(_body, P1, BQ, BR),
        grid=(P1 + P2,),
        in_specs=[pl.BlockSpec((n_skills, d), lambda i: (0, 0))]
        + qc_specs
        + cq_specs,
        out_specs=pl.BlockSpec(
            (LANES * BR, d), lambda i: (jnp.maximum(i - P1, 0), 0)
        ),
        out_shape=jax.ShapeDtypeStruct((n_skills, d), jnp.float32),
        scratch_shapes=[pltpu.VMEM((n_edges, d), jnp.float32)],
    )(skill_embs, *([HG_qc] * LANES), *([HG_cq] * LANES))


# speedup vs baseline: 1.3257x; 1.0745x over previous
import functools

import jax
import jax.numpy as jnp
from jax.experimental import pallas as pl
from jax.experimental.pallas import tpu as pltpu

LANES = 4


def _body(P1, BQ, BR, *refs):
    e_ref = refs[0]
    qc_refs = refs[1:1 + LANES]
    cq_refs = refs[1 + LANES:1 + 2 * LANES]
    out_ref = refs[1 + 2 * LANES]
    msg_ref = refs[2 + 2 * LANES]
    i = pl.program_id(0)

    @pl.when(i < P1)
    def _p1():
        for k in range(LANES):
            acc = jnp.dot(
                qc_refs[k][...].astype(jnp.bfloat16),
                e_ref[...],
                preferred_element_type=jnp.float32,
            )
            msg_ref[pl.ds((i * LANES + k) * BQ, BQ), :] = acc.astype(
                jnp.bfloat16
            )

    @pl.when(i >= P1)
    def _p2():
        for k in range(LANES):
            out_ref[k * BR:(k + 1) * BR, :] = jnp.dot(
                cq_refs[k][...].astype(jnp.bfloat16),
                msg_ref[...],
                preferred_element_type=jnp.float32,
            )


@jax.jit
def kernel(skill_embs, HG_qc, HG_cq):
    n_edges, n_skills = HG_qc.shape
    d = skill_embs.shape[1]
    BQ = 64
    BR = 128
    P1 = n_edges // (BQ * LANES)
    P2 = n_skills // (BR * LANES)

    qc_specs = [
        pl.BlockSpec(
            (BQ, n_skills),
            lambda i, k=k: (jnp.minimum(i, P1 - 1) * LANES + k, 0),
        )
        for k in range(LANES)
    ]
    cq_specs = [
        pl.BlockSpec(
            (BR, n_edges),
            lambda i, k=k: (jnp.maximum(i - P1, 0) * LANES + k, 0),
        )
        for k in range(LANES)
    ]

    return pl.pallas_call(
        functools.partial(_body, P1, BQ, BR),
        grid=(P1 + P2,),
        in_specs=[pl.BlockSpec((n_skills, d), lambda i: (0, 0))]
        + qc_specs
        + cq_specs,
        out_specs=pl.BlockSpec(
            (LANES * BR, d), lambda i: (jnp.maximum(i - P1, 0), 0)
        ),
        out_shape=jax.ShapeDtypeStruct((n_skills, d), jnp.float32),
        scratch_shapes=[pltpu.VMEM((n_edges, d), jnp.bfloat16)],
    )(skill_embs.astype(jnp.bfloat16), *([HG_qc] * LANES), *([HG_cq] * LANES))
